# baseline (speedup 1.0000x reference)
import functools

import jax
import jax.numpy as jnp
from jax.experimental import pallas as pl
from jax.experimental.pallas import tpu as pltpu
from jax.experimental.pallas import tpu_sc as plsc

D_MODEL = 768
NUM_EXPERTS = 8
EXPERT_DIM = 2048
TOP_K = 2
EM_ITERS = 5
LOAD_BALANCE_WEIGHT = 0.1

GB = 128
NB = 71
P_PAD = NB * GB
NW = 32


def _router_body(x_ref, gw_ref, gb_ref, gt_ref,
                 loss_ref, c1_ref, c2_ref, pos1_ref, pos2_ref, be_ref):
    x = x_ref[...]
    T = x.shape[0]
    E = NUM_EXPERTS

    logits = jnp.dot(x, gw_ref[...], preferred_element_type=jnp.float32)
    logits = (logits + gb_ref[...]) / gt_ref[0, 0]

    m = jnp.max(logits, axis=1, keepdims=True)
    ex = jnp.exp(logits - m)
    sm = ex / jnp.sum(ex, axis=1, keepdims=True)

    iota = jax.lax.broadcasted_iota(jnp.int32, (T, E), 1)

    m1 = jnp.max(sm, axis=1, keepdims=True)
    i1 = jnp.min(jnp.where(sm == m1, iota, E), axis=1, keepdims=True)
    sm_masked = jnp.where(iota == i1, -jnp.inf, sm)
    m2 = jnp.max(sm_masked, axis=1, keepdims=True)
    i2 = jnp.min(jnp.where(sm_masked == m2, iota, E), axis=1, keepdims=True)

    oh1 = (iota == i1).astype(jnp.float32)
    oh2 = (iota == i2).astype(jnp.float32)

    usage = jnp.sum(oh1, axis=0, keepdims=True)
    actual = usage / jnp.float32(T) + 1e-8
    actual = actual / jnp.sum(actual)
    unif = jnp.float32(1.0 / E)
    kl = jnp.sum(unif * (jnp.log(unif) - jnp.log(actual)),
                 axis=1, keepdims=True)
    loss_ref[...] = LOAD_BALANCE_WEIGHT * kl

    p = jnp.full((1, E), 1.0 / E, dtype=jnp.float32)
    for _ in range(EM_ITERS):
        ea = sm * p
        ea = ea / (jnp.sum(ea, axis=1, keepdims=True) + 1e-8)
        counts = jnp.sum(ea, axis=0, keepdims=True)
        p = counts / (jnp.sum(counts) + 1e-8)

    bal1 = m1 * jnp.sum(oh1 * p, axis=1, keepdims=True)
    bal2 = m2 * jnp.sum(oh2 * p, axis=1, keepdims=True)
    denom = bal1 + bal2 + 1e-8
    c1_ref[...] = bal1 / denom
    c2_ref[...] = bal2 / denom

    cnt1 = jnp.sum(oh1, axis=0, keepdims=True)
    cnt2 = jnp.sum(oh2, axis=0, keepdims=True)
    cnt = cnt1 + cnt2
    cnt_pad = jnp.ceil(cnt / GB) * GB

    r8 = jax.lax.broadcasted_iota(jnp.int32, (E, E), 0)
    c8 = jax.lax.broadcasted_iota(jnp.int32, (E, E), 1)
    poff = jnp.dot(cnt_pad, (r8 < c8).astype(jnp.float32),
                   preferred_element_type=jnp.float32)

    CH = 512
    tri = (jax.lax.broadcasted_iota(jnp.int32, (CH, CH), 0)
           >= jax.lax.broadcasted_iota(jnp.int32, (CH, CH), 1)
           ).astype(jnp.float32)

    def excl_ranks(oh):
        base = jnp.zeros((1, E), jnp.float32)
        pieces = []
        for ci in range(T // CH):
            chunk = oh[ci * CH:(ci + 1) * CH]
            incl = jnp.dot(tri, chunk,
                           preferred_element_type=jnp.float32) + base
            pieces.append(jnp.sum(chunk * incl, axis=1, keepdims=True) - 1.0)
            base = base + jnp.sum(chunk, axis=0, keepdims=True)
        return jnp.concatenate(pieces, axis=0)

    r1 = excl_ranks(oh1)
    r2 = excl_ranks(oh2) + jnp.sum(oh2 * cnt1, axis=1, keepdims=True)
    pos1 = jnp.sum(oh1 * poff, axis=1, keepdims=True) + r1
    pos2 = jnp.sum(oh2 * poff, axis=1, keepdims=True) + r2
    pos1_ref[...] = pos1.astype(jnp.int32)
    pos2_ref[...] = pos2.astype(jnp.int32)

    bstart = (jax.lax.broadcasted_iota(jnp.int32, (128, 1), 0)
              * GB).astype(jnp.float32)
    nle = jnp.sum((poff <= bstart).astype(jnp.int32), axis=1, keepdims=True)
    be_ref[...] = jnp.clip(nle - 1, 0, E - 1)


def _router(x_flat, gate_w, gate_b, gate_temp):
    T = x_flat.shape[0]
    E = NUM_EXPERTS
    return pl.pallas_call(
        _router_body,
        out_shape=(
            jax.ShapeDtypeStruct((1, 1), jnp.float32),
            jax.ShapeDtypeStruct((T, 1), jnp.float32),
            jax.ShapeDtypeStruct((T, 1), jnp.float32),
            jax.ShapeDtypeStruct((T, 1), jnp.int32),
            jax.ShapeDtypeStruct((T, 1), jnp.int32),
            jax.ShapeDtypeStruct((128, 1), jnp.int32),
        ),
    )(x_flat, gate_w, gate_b.reshape(1, E), gate_temp.reshape(1, 1))


def _sc_scatter(x_flat, pos1, pos2):
    T, D = x_flat.shape
    TPW = T // NW
    mesh = plsc.VectorSubcoreMesh(core_axis_name="c", subcore_axis_name="s",
                                  num_cores=2, num_subcores=16)

    @functools.partial(
        pl.kernel, mesh=mesh,
        out_type=jax.ShapeDtypeStruct((P_PAD, D), jnp.float32),
        scratch_types=[
            pltpu.VMEM((TPW,), jnp.int32),
            pltpu.VMEM((TPW,), jnp.int32),
            pltpu.VMEM((TPW, D), jnp.float32),
            pltpu.SemaphoreType.DMA,
            pltpu.SemaphoreType.DMA,
        ],
    )
    def scatter_kernel(x_hbm, pos1_hbm, pos2_hbm, xs_hbm,
                       idx1_v, idx2_v, rows_v, sem1, sem2):
        wid = jax.lax.axis_index("s") * 2 + jax.lax.axis_index("c")
        base = wid * TPW
        pltpu.sync_copy(pos1_hbm.at[pl.ds(base, TPW)], idx1_v)
        pltpu.sync_copy(pos2_hbm.at[pl.ds(base, TPW)], idx2_v)
        pltpu.sync_copy(x_hbm.at[pl.ds(base, TPW)], rows_v)
        cp1 = pltpu.async_copy(rows_v, xs_hbm.at[idx1_v], sem1)
        cp2 = pltpu.async_copy(rows_v, xs_hbm.at[idx2_v], sem2)
        cp1.wait()
        cp2.wait()

    return scatter_kernel(x_flat, pos1, pos2)


def _sc_gather(y, pos1, pos2):
    T = pos1.shape[0]
    D = y.shape[1]
    TPW = T // NW
    mesh = plsc.VectorSubcoreMesh(core_axis_name="c", subcore_axis_name="s",
                                  num_cores=2, num_subcores=16)

    @functools.partial(
        pl.kernel, mesh=mesh,
        out_type=(
            jax.ShapeDtypeStruct((T, D), jnp.float32),
            jax.ShapeDtypeStruct((T, D), jnp.float32),
        ),
        scratch_types=[
            pltpu.VMEM((TPW,), jnp.int32),
            pltpu.VMEM((TPW, D), jnp.float32),
            pltpu.SemaphoreType.DMA,
        ],
    )
    def gather_kernel(y_hbm, pos1_hbm, pos2_hbm, g1_hbm, g2_hbm,
                      idx_v, rows_v, sem):
        wid = jax.lax.axis_index("s") * 2 + jax.lax.axis_index("c")
        base = wid * TPW
        pltpu.sync_copy(pos1_hbm.at[pl.ds(base, TPW)], idx_v)
        pltpu.async_copy(y_hbm.at[idx_v], rows_v, sem).wait()
        pltpu.sync_copy(rows_v, g1_hbm.at[pl.ds(base, TPW)])
        pltpu.sync_copy(pos2_hbm.at[pl.ds(base, TPW)], idx_v)
        pltpu.async_copy(y_hbm.at[idx_v], rows_v, sem).wait()
        pltpu.sync_copy(rows_v, g2_hbm.at[pl.ds(base, TPW)])

    return gather_kernel(y, pos1, pos2)


def _group_body(be_ref, xs_ref, w1_ref, b1_ref, w2_ref, b2_ref, y_ref):
    h = jnp.maximum(
        jnp.dot(xs_ref[...], w1_ref[0], preferred_element_type=jnp.float32)
        + b1_ref[0], 0.0)
    y_ref[...] = (jnp.dot(h, w2_ref[0], preferred_element_type=jnp.float32)
                  + b2_ref[0])


def _grouped_ffn(be, xs, w1, b1, w2, b2):
    D, F, E = D_MODEL, EXPERT_DIM, NUM_EXPERTS
    grid_spec = pltpu.PrefetchScalarGridSpec(
        num_scalar_prefetch=1,
        grid=(NB,),
        in_specs=[
            pl.BlockSpec((GB, D), lambda b, be: (b, 0)),
            pl.BlockSpec((1, D, F), lambda b, be: (be[b], 0, 0)),
            pl.BlockSpec((1, 1, F), lambda b, be: (be[b], 0, 0)),
            pl.BlockSpec((1, F, D), lambda b, be: (be[b], 0, 0)),
            pl.BlockSpec((1, 1, D), lambda b, be: (be[b], 0, 0)),
        ],
        out_specs=pl.BlockSpec((GB, D), lambda b, be: (b, 0)),
    )
    return pl.pallas_call(
        _group_body,
        grid_spec=grid_spec,
        out_shape=jax.ShapeDtypeStruct((P_PAD, D), jnp.float32),
    )(be, xs, w1, b1.reshape(E, 1, F), w2, b2.reshape(E, 1, D))


def _shared_body(x_ref, ws1_ref, bs1_ref, ws2_ref, bs2_ref, sh_ref):
    hs = jnp.maximum(
        jnp.dot(x_ref[...], ws1_ref[...], preferred_element_type=jnp.float32)
        + bs1_ref[...], 0.0)
    sh_ref[...] = (jnp.dot(hs, ws2_ref[...], preferred_element_type=jnp.float32)
                   + bs2_ref[...])


def _shared_ffn(x_flat, ws1, bs1, ws2, bs2):
    T, D = x_flat.shape
    F = EXPERT_DIM
    TB = 512
    return pl.pallas_call(
        _shared_body,
        grid=(T // TB,),
        in_specs=[
            pl.BlockSpec((TB, D), lambda t: (t, 0)),
            pl.BlockSpec((D, F), lambda t: (0, 0)),
            pl.BlockSpec((1, F), lambda t: (0, 0)),
            pl.BlockSpec((F, D), lambda t: (0, 0)),
            pl.BlockSpec((1, D), lambda t: (0, 0)),
        ],
        out_specs=pl.BlockSpec((TB, D), lambda t: (t, 0)),
        out_shape=jax.ShapeDtypeStruct((T, D), jnp.float32),
    )(x_flat, ws1, bs1.reshape(1, F), ws2, bs2.reshape(1, D))


def _combine_body(g1_ref, g2_ref, c1_ref, c2_ref, sh_ref, out_ref):
    out_ref[...] = (c1_ref[...] * g1_ref[...] + c2_ref[...] * g2_ref[...]
                    + sh_ref[...])


def _combine(g1, g2, c1, c2, sh):
    T, D = g1.shape
    TB = 1024
    return pl.pallas_call(
        _combine_body,
        grid=(T // TB,),
        in_specs=[
            pl.BlockSpec((TB, D), lambda t: (t, 0)),
            pl.BlockSpec((TB, D), lambda t: (t, 0)),
            pl.BlockSpec((TB, 1), lambda t: (t, 0)),
            pl.BlockSpec((TB, 1), lambda t: (t, 0)),
            pl.BlockSpec((TB, D), lambda t: (t, 0)),
        ],
        out_specs=pl.BlockSpec((TB, D), lambda t: (t, 0)),
        out_shape=jax.ShapeDtypeStruct((T, D), jnp.float32),
    )(g1, g2, c1, c2, sh)


@jax.jit
def kernel(x, gate_w, gate_b, gate_temp, w1, b1, w2, b2, ws1, bs1, ws2, bs2):
    B, S, D = x.shape
    T = B * S
    x_flat = x.reshape(T, D)

    loss, c1, c2, pos1, pos2, be = _router(x_flat, gate_w, gate_b, gate_temp)
    pos1f = pos1.reshape(T)
    pos2f = pos2.reshape(T)
    be_nb = be.reshape(128)[:NB]

    xs = _sc_scatter(x_flat, pos1f, pos2f)
    sh = _shared_ffn(x_flat, ws1, bs1, ws2, bs2)
    y = _grouped_ffn(be_nb, xs, w1, b1, w2, b2)
    g1, g2 = _sc_gather(y, pos1f, pos2f)
    out = _combine(g1, g2, c1, c2, sh)

    return out.reshape(B, S, D), loss.reshape(())


# speedup vs baseline: 1.0097x; 1.0097x over previous
import functools

import jax
import jax.numpy as jnp
from jax.experimental import pallas as pl
from jax.experimental.pallas import tpu as pltpu
from jax.experimental.pallas import tpu_sc as plsc

D_MODEL = 768
NUM_EXPERTS = 8
EXPERT_DIM = 2048
TOP_K = 2
EM_ITERS = 5
LOAD_BALANCE_WEIGHT = 0.1

GB = 128
NB = 71
P_PAD = NB * GB
NW = 32


def _router_body(x_ref, gw_ref, gb_ref, gt_ref,
                 loss_ref, c1_ref, c2_ref, pos1_ref, pos2_ref, be_ref):
    x = x_ref[...]
    T = x.shape[0]
    E = NUM_EXPERTS

    logits = jnp.dot(x, gw_ref[...], preferred_element_type=jnp.float32)
    logits = (logits + gb_ref[...]) / gt_ref[0, 0]

    m = jnp.max(logits, axis=1, keepdims=True)
    ex = jnp.exp(logits - m)
    sm = ex / jnp.sum(ex, axis=1, keepdims=True)

    iota = jax.lax.broadcasted_iota(jnp.int32, (T, E), 1)

    m1 = jnp.max(sm, axis=1, keepdims=True)
    i1 = jnp.min(jnp.where(sm == m1, iota, E), axis=1, keepdims=True)
    sm_masked = jnp.where(iota == i1, -jnp.inf, sm)
    m2 = jnp.max(sm_masked, axis=1, keepdims=True)
    i2 = jnp.min(jnp.where(sm_masked == m2, iota, E), axis=1, keepdims=True)

    oh1 = (iota == i1).astype(jnp.float32)
    oh2 = (iota == i2).astype(jnp.float32)

    usage = jnp.sum(oh1, axis=0, keepdims=True)
    actual = usage / jnp.float32(T) + 1e-8
    actual = actual / jnp.sum(actual)
    unif = jnp.float32(1.0 / E)
    kl = jnp.sum(unif * (jnp.log(unif) - jnp.log(actual)),
                 axis=1, keepdims=True)
    loss_ref[...] = LOAD_BALANCE_WEIGHT * kl

    p = jnp.full((1, E), 1.0 / E, dtype=jnp.float32)
    for _ in range(EM_ITERS):
        ea = sm * p
        ea = ea / (jnp.sum(ea, axis=1, keepdims=True) + 1e-8)
        counts = jnp.sum(ea, axis=0, keepdims=True)
        p = counts / (jnp.sum(counts) + 1e-8)

    bal1 = m1 * jnp.sum(oh1 * p, axis=1, keepdims=True)
    bal2 = m2 * jnp.sum(oh2 * p, axis=1, keepdims=True)
    denom = bal1 + bal2 + 1e-8
    c1_ref[...] = bal1 / denom
    c2_ref[...] = bal2 / denom

    cnt1 = jnp.sum(oh1, axis=0, keepdims=True)
    cnt2 = jnp.sum(oh2, axis=0, keepdims=True)
    cnt = cnt1 + cnt2
    cnt_pad = jnp.ceil(cnt / GB) * GB

    r8 = jax.lax.broadcasted_iota(jnp.int32, (E, E), 0)
    c8 = jax.lax.broadcasted_iota(jnp.int32, (E, E), 1)
    poff = jnp.dot(cnt_pad, (r8 < c8).astype(jnp.float32),
                   preferred_element_type=jnp.float32)

    CH = 512
    tri = (jax.lax.broadcasted_iota(jnp.int32, (CH, CH), 0)
           >= jax.lax.broadcasted_iota(jnp.int32, (CH, CH), 1)
           ).astype(jnp.float32)

    def excl_ranks(oh):
        base = jnp.zeros((1, E), jnp.float32)
        pieces = []
        for ci in range(T // CH):
            chunk = oh[ci * CH:(ci + 1) * CH]
            incl = jnp.dot(tri, chunk,
                           preferred_element_type=jnp.float32) + base
            pieces.append(jnp.sum(chunk * incl, axis=1, keepdims=True) - 1.0)
            base = base + jnp.sum(chunk, axis=0, keepdims=True)
        return jnp.concatenate(pieces, axis=0)

    r1 = excl_ranks(oh1)
    r2 = excl_ranks(oh2) + jnp.sum(oh2 * cnt1, axis=1, keepdims=True)
    pos1 = jnp.sum(oh1 * poff, axis=1, keepdims=True) + r1
    pos2 = jnp.sum(oh2 * poff, axis=1, keepdims=True) + r2
    pos1_ref[...] = pos1.astype(jnp.int32)
    pos2_ref[...] = pos2.astype(jnp.int32)

    bstart = (jax.lax.broadcasted_iota(jnp.int32, (128, 1), 0)
              * GB).astype(jnp.float32)
    nle = jnp.sum((poff <= bstart).astype(jnp.int32), axis=1, keepdims=True)
    be_ref[...] = jnp.clip(nle - 1, 0, E - 1)


def _router(x_flat, gate_w, gate_b, gate_temp):
    T = x_flat.shape[0]
    E = NUM_EXPERTS
    return pl.pallas_call(
        _router_body,
        out_shape=(
            jax.ShapeDtypeStruct((1, 1), jnp.float32),
            jax.ShapeDtypeStruct((T, 1), jnp.float32),
            jax.ShapeDtypeStruct((T, 1), jnp.float32),
            jax.ShapeDtypeStruct((T, 1), jnp.int32),
            jax.ShapeDtypeStruct((T, 1), jnp.int32),
            jax.ShapeDtypeStruct((128, 1), jnp.int32),
        ),
    )(x_flat, gate_w, gate_b.reshape(1, E), gate_temp.reshape(1, 1))


def _sc_scatter(x_flat, pos1, pos2):
    T, D = x_flat.shape
    TPW = T // NW
    mesh = plsc.VectorSubcoreMesh(core_axis_name="c", subcore_axis_name="s",
                                  num_cores=2, num_subcores=16)

    @functools.partial(
        pl.kernel, mesh=mesh,
        out_type=jax.ShapeDtypeStruct((P_PAD, D), jnp.float32),
        scratch_types=[
            pltpu.VMEM((TPW,), jnp.int32),
            pltpu.VMEM((TPW,), jnp.int32),
            pltpu.VMEM((TPW, D), jnp.float32),
            pltpu.SemaphoreType.DMA,
            pltpu.SemaphoreType.DMA,
        ],
    )
    def scatter_kernel(x_hbm, pos1_hbm, pos2_hbm, xs_hbm,
                       idx1_v, idx2_v, rows_v, sem1, sem2):
        wid = jax.lax.axis_index("s") * 2 + jax.lax.axis_index("c")
        base = wid * TPW
        pltpu.sync_copy(pos1_hbm.at[pl.ds(base, TPW)], idx1_v)
        pltpu.sync_copy(pos2_hbm.at[pl.ds(base, TPW)], idx2_v)
        pltpu.sync_copy(x_hbm.at[pl.ds(base, TPW)], rows_v)
        cp1 = pltpu.async_copy(rows_v, xs_hbm.at[idx1_v], sem1)
        cp2 = pltpu.async_copy(rows_v, xs_hbm.at[idx2_v], sem2)
        cp1.wait()
        cp2.wait()

    return scatter_kernel(x_flat, pos1, pos2)


def _sc_gather(y, pos1, pos2):
    T = pos1.shape[0]
    D = y.shape[1]
    TPW = T // NW
    mesh = plsc.VectorSubcoreMesh(core_axis_name="c", subcore_axis_name="s",
                                  num_cores=2, num_subcores=16)

    @functools.partial(
        pl.kernel, mesh=mesh,
        out_type=(
            jax.ShapeDtypeStruct((T, D), jnp.float32),
            jax.ShapeDtypeStruct((T, D), jnp.float32),
        ),
        scratch_types=[
            pltpu.VMEM((TPW,), jnp.int32),
            pltpu.VMEM((TPW, D), jnp.float32),
            pltpu.SemaphoreType.DMA,
        ],
    )
    def gather_kernel(y_hbm, pos1_hbm, pos2_hbm, g1_hbm, g2_hbm,
                      idx_v, rows_v, sem):
        wid = jax.lax.axis_index("s") * 2 + jax.lax.axis_index("c")
        base = wid * TPW
        pltpu.sync_copy(pos1_hbm.at[pl.ds(base, TPW)], idx_v)
        pltpu.async_copy(y_hbm.at[idx_v], rows_v, sem).wait()
        pltpu.sync_copy(rows_v, g1_hbm.at[pl.ds(base, TPW)])
        pltpu.sync_copy(pos2_hbm.at[pl.ds(base, TPW)], idx_v)
        pltpu.async_copy(y_hbm.at[idx_v], rows_v, sem).wait()
        pltpu.sync_copy(rows_v, g2_hbm.at[pl.ds(base, TPW)])

    return gather_kernel(y, pos1, pos2)


def _group_body(be_ref, xs_ref, w1_ref, b1_ref, w2_ref, b2_ref, y_ref):
    h = jnp.maximum(
        jnp.dot(xs_ref[...], w1_ref[0], preferred_element_type=jnp.float32)
        + b1_ref[0], 0.0)
    y_ref[...] = (jnp.dot(h, w2_ref[0], preferred_element_type=jnp.float32)
                  + b2_ref[0])


def _grouped_ffn(be, xs, w1, b1, w2, b2):
    D, F, E = D_MODEL, EXPERT_DIM, NUM_EXPERTS
    grid_spec = pltpu.PrefetchScalarGridSpec(
        num_scalar_prefetch=1,
        grid=(NB,),
        in_specs=[
            pl.BlockSpec((GB, D), lambda b, be: (b, 0)),
            pl.BlockSpec((1, D, F), lambda b, be: (be[b], 0, 0)),
            pl.BlockSpec((1, 1, F), lambda b, be: (be[b], 0, 0)),
            pl.BlockSpec((1, F, D), lambda b, be: (be[b], 0, 0)),
            pl.BlockSpec((1, 1, D), lambda b, be: (be[b], 0, 0)),
        ],
        out_specs=pl.BlockSpec((GB, D), lambda b, be: (b, 0)),
    )
    return pl.pallas_call(
        _group_body,
        grid_spec=grid_spec,
        out_shape=jax.ShapeDtypeStruct((P_PAD, D), jnp.float32),
    )(be, xs, w1, b1.reshape(E, 1, F), w2, b2.reshape(E, 1, D))


def _shared_body(x_ref, ws1_ref, bs1_ref, ws2_ref, bs2_ref, sh_ref):
    hs = jnp.maximum(
        jnp.dot(x_ref[...], ws1_ref[...], preferred_element_type=jnp.float32)
        + bs1_ref[...], 0.0)
    sh_ref[...] = (jnp.dot(hs, ws2_ref[...], preferred_element_type=jnp.float32)
                   + bs2_ref[...])


def _shared_ffn(x_flat, ws1, bs1, ws2, bs2):
    T, D = x_flat.shape
    F = EXPERT_DIM
    TB = 512
    return pl.pallas_call(
        _shared_body,
        grid=(T // TB,),
        in_specs=[
            pl.BlockSpec((TB, D), lambda t: (t, 0)),
            pl.BlockSpec((D, F), lambda t: (0, 0)),
            pl.BlockSpec((1, F), lambda t: (0, 0)),
            pl.BlockSpec((F, D), lambda t: (0, 0)),
            pl.BlockSpec((1, D), lambda t: (0, 0)),
        ],
        out_specs=pl.BlockSpec((TB, D), lambda t: (t, 0)),
        out_shape=jax.ShapeDtypeStruct((T, D), jnp.float32),
    )(x_flat, ws1, bs1.reshape(1, F), ws2, bs2.reshape(1, D))


def _combine_body(g1_ref, g2_ref, c1_ref, c2_ref, x_ref,
                  ws1_ref, bs1_ref, ws2_ref, bs2_ref, out_ref):
    hs = jnp.maximum(
        jnp.dot(x_ref[...], ws1_ref[...], preferred_element_type=jnp.float32)
        + bs1_ref[...], 0.0)
    shared = (jnp.dot(hs, ws2_ref[...], preferred_element_type=jnp.float32)
              + bs2_ref[...])
    out_ref[...] = (c1_ref[...] * g1_ref[...] + c2_ref[...] * g2_ref[...]
                    + shared)


def _combine(g1, g2, c1, c2, x_flat, ws1, bs1, ws2, bs2):
    T, D = x_flat.shape
    F = EXPERT_DIM
    TB = 512
    return pl.pallas_call(
        _combine_body,
        grid=(T // TB,),
        in_specs=[
            pl.BlockSpec((TB, D), lambda t: (t, 0)),
            pl.BlockSpec((TB, D), lambda t: (t, 0)),
            pl.BlockSpec((TB, 1), lambda t: (t, 0)),
            pl.BlockSpec((TB, 1), lambda t: (t, 0)),
            pl.BlockSpec((TB, D), lambda t: (t, 0)),
            pl.BlockSpec((D, F), lambda t: (0, 0)),
            pl.BlockSpec((1, F), lambda t: (0, 0)),
            pl.BlockSpec((F, D), lambda t: (0, 0)),
            pl.BlockSpec((1, D), lambda t: (0, 0)),
        ],
        out_specs=pl.BlockSpec((TB, D), lambda t: (t, 0)),
        out_shape=jax.ShapeDtypeStruct((T, D), jnp.float32),
    )(g1, g2, c1, c2, x_flat, ws1, bs1.reshape(1, F), ws2, bs2.reshape(1, D))


@jax.jit
def kernel(x, gate_w, gate_b, gate_temp, w1, b1, w2, b2, ws1, bs1, ws2, bs2):
    B, S, D = x.shape
    T = B * S
    x_flat = x.reshape(T, D)

    loss, c1, c2, pos1, pos2, be = _router(x_flat, gate_w, gate_b, gate_temp)
    pos1f = pos1.reshape(T)
    pos2f = pos2.reshape(T)
    be_nb = be.reshape(128)[:NB]

    xs = _sc_scatter(x_flat, pos1f, pos2f)
    y = _grouped_ffn(be_nb, xs, w1, b1, w2, b2)
    g1, g2 = _sc_gather(y, pos1f, pos2f)
    out = _combine(g1, g2, c1, c2, x_flat, ws1, bs1, ws2, bs2)

    return out.reshape(B, S, D), loss.reshape(())


# speedup vs baseline: 1.0714x; 1.0611x over previous
import functools

import jax
import jax.numpy as jnp
from jax.experimental import pallas as pl
from jax.experimental.pallas import tpu as pltpu
from jax.experimental.pallas import tpu_sc as plsc

D_MODEL = 768
NUM_EXPERTS = 8
EXPERT_DIM = 2048
TOP_K = 2
EM_ITERS = 5
LOAD_BALANCE_WEIGHT = 0.1

GB = 256
NB = 39
P_PAD = NB * GB
NW = 32


def _router_body(x_ref, gw_ref, gb_ref, gt_ref,
                 loss_ref, c1_ref, c2_ref, pos1_ref, pos2_ref, be_ref):
    x = x_ref[...]
    T = x.shape[0]
    E = NUM_EXPERTS

    logits = jnp.dot(x, gw_ref[...], preferred_element_type=jnp.float32)
    logits = (logits + gb_ref[...]) / gt_ref[0, 0]

    m = jnp.max(logits, axis=1, keepdims=True)
    ex = jnp.exp(logits - m)
    sm = ex / jnp.sum(ex, axis=1, keepdims=True)

    iota = jax.lax.broadcasted_iota(jnp.int32, (T, E), 1)

    m1 = jnp.max(sm, axis=1, keepdims=True)
    i1 = jnp.min(jnp.where(sm == m1, iota, E), axis=1, keepdims=True)
    sm_masked = jnp.where(iota == i1, -jnp.inf, sm)
    m2 = jnp.max(sm_masked, axis=1, keepdims=True)
    i2 = jnp.min(jnp.where(sm_masked == m2, iota, E), axis=1, keepdims=True)

    oh1 = (iota == i1).astype(jnp.float32)
    oh2 = (iota == i2).astype(jnp.float32)

    usage = jnp.sum(oh1, axis=0, keepdims=True)
    actual = usage / jnp.float32(T) + 1e-8
    actual = actual / jnp.sum(actual)
    unif = jnp.float32(1.0 / E)
    kl = jnp.sum(unif * (jnp.log(unif) - jnp.log(actual)),
                 axis=1, keepdims=True)
    loss_ref[...] = LOAD_BALANCE_WEIGHT * kl

    p = jnp.full((1, E), 1.0 / E, dtype=jnp.float32)
    for _ in range(EM_ITERS):
        rs = jnp.dot(sm, p.reshape(E, 1),
                     preferred_element_type=jnp.float32)
        inv = 1.0 / (rs + 1e-8)
        counts = p * jnp.sum(sm * inv, axis=0, keepdims=True)
        p = counts / (jnp.sum(counts) + 1e-8)

    bal1 = m1 * jnp.sum(oh1 * p, axis=1, keepdims=True)
    bal2 = m2 * jnp.sum(oh2 * p, axis=1, keepdims=True)
    denom = bal1 + bal2 + 1e-8
    c1_ref[...] = bal1 / denom
    c2_ref[...] = bal2 / denom

    cnt1 = jnp.sum(oh1, axis=0, keepdims=True)
    cnt2 = jnp.sum(oh2, axis=0, keepdims=True)
    cnt = cnt1 + cnt2
    cnt_pad = jnp.ceil(cnt / GB) * GB

    r8 = jax.lax.broadcasted_iota(jnp.int32, (E, E), 0)
    c8 = jax.lax.broadcasted_iota(jnp.int32, (E, E), 1)
    poff = jnp.dot(cnt_pad, (r8 < c8).astype(jnp.float32),
                   preferred_element_type=jnp.float32)

    CH = 512
    tri = (jax.lax.broadcasted_iota(jnp.int32, (CH, CH), 0)
           >= jax.lax.broadcasted_iota(jnp.int32, (CH, CH), 1)
           ).astype(jnp.float32)

    ohj = jnp.concatenate([oh1, oh2], axis=1)
    base = jnp.zeros((1, 2 * E), jnp.float32)
    p1, p2 = [], []
    for ci in range(T // CH):
        chunk = ohj[ci * CH:(ci + 1) * CH]
        incl = jnp.dot(tri, chunk, preferred_element_type=jnp.float32) + base
        sel = chunk * incl
        p1.append(jnp.sum(sel[:, :E], axis=1, keepdims=True) - 1.0)
        p2.append(jnp.sum(sel[:, E:], axis=1, keepdims=True) - 1.0)
        base = base + jnp.sum(chunk, axis=0, keepdims=True)

    r1 = jnp.concatenate(p1, axis=0)
    r2 = (jnp.concatenate(p2, axis=0)
          + jnp.sum(oh2 * cnt1, axis=1, keepdims=True))
    pos1 = jnp.sum(oh1 * poff, axis=1, keepdims=True) + r1
    pos2 = jnp.sum(oh2 * poff, axis=1, keepdims=True) + r2
    pos1_ref[...] = pos1.astype(jnp.int32)
    pos2_ref[...] = pos2.astype(jnp.int32)

    bstart = (jax.lax.broadcasted_iota(jnp.int32, (128, 1), 0)
              * GB).astype(jnp.float32)
    nle = jnp.sum((poff <= bstart).astype(jnp.int32), axis=1, keepdims=True)
    be_ref[...] = jnp.clip(nle - 1, 0, E - 1)


def _router(x_flat, gate_w, gate_b, gate_temp):
    T = x_flat.shape[0]
    E = NUM_EXPERTS
    return pl.pallas_call(
        _router_body,
        out_shape=(
            jax.ShapeDtypeStruct((1, 1), jnp.float32),
            jax.ShapeDtypeStruct((T, 1), jnp.float32),
            jax.ShapeDtypeStruct((T, 1), jnp.float32),
            jax.ShapeDtypeStruct((T, 1), jnp.int32),
            jax.ShapeDtypeStruct((T, 1), jnp.int32),
            jax.ShapeDtypeStruct((128, 1), jnp.int32),
        ),
    )(x_flat, gate_w, gate_b.reshape(1, E), gate_temp.reshape(1, 1))


def _sc_scatter(x_flat, pos1, pos2):
    T, D = x_flat.shape
    TPW = T // NW
    mesh = plsc.VectorSubcoreMesh(core_axis_name="c", subcore_axis_name="s",
                                  num_cores=2, num_subcores=16)

    @functools.partial(
        pl.kernel, mesh=mesh,
        out_type=jax.ShapeDtypeStruct((P_PAD, D), jnp.float32),
        scratch_types=[
            pltpu.VMEM((TPW,), jnp.int32),
            pltpu.VMEM((TPW,), jnp.int32),
            pltpu.VMEM((TPW, D), jnp.float32),
            pltpu.SemaphoreType.DMA,
            pltpu.SemaphoreType.DMA,
        ],
    )
    def scatter_kernel(x_hbm, pos1_hbm, pos2_hbm, xs_hbm,
                       idx1_v, idx2_v, rows_v, sem1, sem2):
        wid = jax.lax.axis_index("s") * 2 + jax.lax.axis_index("c")
        base = wid * TPW
        pltpu.sync_copy(pos1_hbm.at[pl.ds(base, TPW)], idx1_v)
        pltpu.sync_copy(pos2_hbm.at[pl.ds(base, TPW)], idx2_v)
        pltpu.sync_copy(x_hbm.at[pl.ds(base, TPW)], rows_v)
        cp1 = pltpu.async_copy(rows_v, xs_hbm.at[idx1_v], sem1)
        cp2 = pltpu.async_copy(rows_v, xs_hbm.at[idx2_v], sem2)
        cp1.wait()
        cp2.wait()

    return scatter_kernel(x_flat, pos1, pos2)


def _sc_gather(y, pos1, pos2):
    T = pos1.shape[0]
    D = y.shape[1]
    TPW = T // NW
    mesh = plsc.VectorSubcoreMesh(core_axis_name="c", subcore_axis_name="s",
                                  num_cores=2, num_subcores=16)

    @functools.partial(
        pl.kernel, mesh=mesh,
        out_type=(
            jax.ShapeDtypeStruct((T, D), jnp.float32),
            jax.ShapeDtypeStruct((T, D), jnp.float32),
        ),
        scratch_types=[
            pltpu.VMEM((TPW,), jnp.int32),
            pltpu.VMEM((TPW, D), jnp.float32),
            pltpu.SemaphoreType.DMA,
        ],
    )
    def gather_kernel(y_hbm, pos1_hbm, pos2_hbm, g1_hbm, g2_hbm,
                      idx_v, rows_v, sem):
        wid = jax.lax.axis_index("s") * 2 + jax.lax.axis_index("c")
        base = wid * TPW
        pltpu.sync_copy(pos1_hbm.at[pl.ds(base, TPW)], idx_v)
        pltpu.async_copy(y_hbm.at[idx_v], rows_v, sem).wait()
        pltpu.sync_copy(rows_v, g1_hbm.at[pl.ds(base, TPW)])
        pltpu.sync_copy(pos2_hbm.at[pl.ds(base, TPW)], idx_v)
        pltpu.async_copy(y_hbm.at[idx_v], rows_v, sem).wait()
        pltpu.sync_copy(rows_v, g2_hbm.at[pl.ds(base, TPW)])

    return gather_kernel(y, pos1, pos2)


def _group_body(be_ref, xs_ref, w1_ref, b1_ref, w2_ref, b2_ref, y_ref):
    h = jnp.maximum(
        jnp.dot(xs_ref[...], w1_ref[0], preferred_element_type=jnp.float32)
        + b1_ref[0], 0.0)
    y_ref[...] = (jnp.dot(h, w2_ref[0], preferred_element_type=jnp.float32)
                  + b2_ref[0])


def _grouped_ffn(be, xs, w1, b1, w2, b2):
    D, F, E = D_MODEL, EXPERT_DIM, NUM_EXPERTS
    grid_spec = pltpu.PrefetchScalarGridSpec(
        num_scalar_prefetch=1,
        grid=(NB,),
        in_specs=[
            pl.BlockSpec((GB, D), lambda b, be: (b, 0)),
            pl.BlockSpec((1, D, F), lambda b, be: (be[b], 0, 0)),
            pl.BlockSpec((1, 1, F), lambda b, be: (be[b], 0, 0)),
            pl.BlockSpec((1, F, D), lambda b, be: (be[b], 0, 0)),
            pl.BlockSpec((1, 1, D), lambda b, be: (be[b], 0, 0)),
        ],
        out_specs=pl.BlockSpec((GB, D), lambda b, be: (b, 0)),
    )
    return pl.pallas_call(
        _group_body,
        grid_spec=grid_spec,
        out_shape=jax.ShapeDtypeStruct((P_PAD, D), jnp.float32),
    )(be, xs, w1, b1.reshape(E, 1, F), w2, b2.reshape(E, 1, D))


def _shared_body(x_ref, ws1_ref, bs1_ref, ws2_ref, bs2_ref, sh_ref):
    hs = jnp.maximum(
        jnp.dot(x_ref[...], ws1_ref[...], preferred_element_type=jnp.float32)
        + bs1_ref[...], 0.0)
    sh_ref[...] = (jnp.dot(hs, ws2_ref[...], preferred_element_type=jnp.float32)
                   + bs2_ref[...])


def _shared_ffn(x_flat, ws1, bs1, ws2, bs2):
    T, D = x_flat.shape
    F = EXPERT_DIM
    TB = 512
    return pl.pallas_call(
        _shared_body,
        grid=(T // TB,),
        in_specs=[
            pl.BlockSpec((TB, D), lambda t: (t, 0)),
            pl.BlockSpec((D, F), lambda t: (0, 0)),
            pl.BlockSpec((1, F), lambda t: (0, 0)),
            pl.BlockSpec((F, D), lambda t: (0, 0)),
            pl.BlockSpec((1, D), lambda t: (0, 0)),
        ],
        out_specs=pl.BlockSpec((TB, D), lambda t: (t, 0)),
        out_shape=jax.ShapeDtypeStruct((T, D), jnp.float32),
    )(x_flat, ws1, bs1.reshape(1, F), ws2, bs2.reshape(1, D))


def _combine_body(g1_ref, g2_ref, c1_ref, c2_ref, x_ref,
                  ws1_ref, bs1_ref, ws2_ref, bs2_ref, out_ref):
    hs = jnp.maximum(
        jnp.dot(x_ref[...], ws1_ref[...], preferred_element_type=jnp.float32)
        + bs1_ref[...], 0.0)
    shared = (jnp.dot(hs, ws2_ref[...], preferred_element_type=jnp.float32)
              + bs2_ref[...])
    out_ref[...] = (c1_ref[...] * g1_ref[...] + c2_ref[...] * g2_ref[...]
                    + shared)


def _combine(g1, g2, c1, c2, x_flat, ws1, bs1, ws2, bs2):
    T, D = x_flat.shape
    F = EXPERT_DIM
    TB = 512
    return pl.pallas_call(
        _combine_body,
        grid=(T // TB,),
        in_specs=[
            pl.BlockSpec((TB, D), lambda t: (t, 0)),
            pl.BlockSpec((TB, D), lambda t: (t, 0)),
            pl.BlockSpec((TB, 1), lambda t: (t, 0)),
            pl.BlockSpec((TB, 1), lambda t: (t, 0)),
            pl.BlockSpec((TB, D), lambda t: (t, 0)),
            pl.BlockSpec((D, F), lambda t: (0, 0)),
            pl.BlockSpec((1, F), lambda t: (0, 0)),
            pl.BlockSpec((F, D), lambda t: (0, 0)),
            pl.BlockSpec((1, D), lambda t: (0, 0)),
        ],
        out_specs=pl.BlockSpec((TB, D), lambda t: (t, 0)),
        out_shape=jax.ShapeDtypeStruct((T, D), jnp.float32),
    )(g1, g2, c1, c2, x_flat, ws1, bs1.reshape(1, F), ws2, bs2.reshape(1, D))


@jax.jit
def kernel(x, gate_w, gate_b, gate_temp, w1, b1, w2, b2, ws1, bs1, ws2, bs2):
    B, S, D = x.shape
    T = B * S
    x_flat = x.reshape(T, D)

    loss, c1, c2, pos1, pos2, be = _router(x_flat, gate_w, gate_b, gate_temp)
    pos1f = pos1.reshape(T)
    pos2f = pos2.reshape(T)
    be_nb = be.reshape(128)[:NB]

    xs = _sc_scatter(x_flat, pos1f, pos2f)
    y = _grouped_ffn(be_nb, xs, w1, b1, w2, b2)
    g1, g2 = _sc_gather(y, pos1f, pos2f)
    out = _combine(g1, g2, c1, c2, x_flat, ws1, bs1, ws2, bs2)

    return out.reshape(B, S, D), loss.reshape(())


# speedup vs baseline: 1.1362x; 1.0605x over previous
import functools

import jax
import jax.numpy as jnp
from jax.experimental import pallas as pl
from jax.experimental.pallas import tpu as pltpu
from jax.experimental.pallas import tpu_sc as plsc

D_MODEL = 768
NUM_EXPERTS = 8
EXPERT_DIM = 2048
TOP_K = 2
EM_ITERS = 5
LOAD_BALANCE_WEIGHT = 0.1

GB = 256
NB = 39
P_PAD = NB * GB
NW = 32


def _router_body(x_ref, gw_ref, gb_ref, gt_ref,
                 loss_ref, c1_ref, c2_ref, pos1_ref, pos2_ref, be_ref):
    x = x_ref[...]
    T = x.shape[0]
    E = NUM_EXPERTS

    logits = jnp.dot(x, gw_ref[...], preferred_element_type=jnp.float32)
    logits = (logits + gb_ref[...]) / gt_ref[0, 0]

    m = jnp.max(logits, axis=1, keepdims=True)
    ex = jnp.exp(logits - m)
    sm = ex / jnp.sum(ex, axis=1, keepdims=True)

    iota = jax.lax.broadcasted_iota(jnp.int32, (T, E), 1)

    m1 = jnp.max(sm, axis=1, keepdims=True)
    i1 = jnp.min(jnp.where(sm == m1, iota, E), axis=1, keepdims=True)
    sm_masked = jnp.where(iota == i1, -jnp.inf, sm)
    m2 = jnp.max(sm_masked, axis=1, keepdims=True)
    i2 = jnp.min(jnp.where(sm_masked == m2, iota, E), axis=1, keepdims=True)

    oh1 = (iota == i1).astype(jnp.float32)
    oh2 = (iota == i2).astype(jnp.float32)

    usage = jnp.sum(oh1, axis=0, keepdims=True)
    actual = usage / jnp.float32(T) + 1e-8
    actual = actual / jnp.sum(actual)
    unif = jnp.float32(1.0 / E)
    kl = jnp.sum(unif * (jnp.log(unif) - jnp.log(actual)),
                 axis=1, keepdims=True)
    loss_ref[...] = LOAD_BALANCE_WEIGHT * kl

    p = jnp.full((1, E), 1.0 / E, dtype=jnp.float32)
    for _ in range(EM_ITERS):
        rs = jnp.dot(sm, p.reshape(E, 1),
                     preferred_element_type=jnp.float32)
        inv = 1.0 / (rs + 1e-8)
        counts = p * jnp.sum(sm * inv, axis=0, keepdims=True)
        p = counts / (jnp.sum(counts) + 1e-8)

    bal1 = m1 * jnp.sum(oh1 * p, axis=1, keepdims=True)
    bal2 = m2 * jnp.sum(oh2 * p, axis=1, keepdims=True)
    denom = bal1 + bal2 + 1e-8
    c1_ref[...] = bal1 / denom
    c2_ref[...] = bal2 / denom

    cnt1 = jnp.sum(oh1, axis=0, keepdims=True)
    cnt2 = jnp.sum(oh2, axis=0, keepdims=True)
    cnt = cnt1 + cnt2
    cnt_pad = jnp.ceil(cnt / GB) * GB

    r8 = jax.lax.broadcasted_iota(jnp.int32, (E, E), 0)
    c8 = jax.lax.broadcasted_iota(jnp.int32, (E, E), 1)
    poff = jnp.dot(cnt_pad, (r8 < c8).astype(jnp.float32),
                   preferred_element_type=jnp.float32)

    CH = 512
    tri = (jax.lax.broadcasted_iota(jnp.int32, (CH, CH), 0)
           >= jax.lax.broadcasted_iota(jnp.int32, (CH, CH), 1)
           ).astype(jnp.float32)

    ohj = jnp.concatenate([oh1, oh2], axis=1)
    base = jnp.zeros((1, 2 * E), jnp.float32)
    p1, p2 = [], []
    for ci in range(T // CH):
        chunk = ohj[ci * CH:(ci + 1) * CH]
        incl = jnp.dot(tri, chunk, preferred_element_type=jnp.float32) + base
        sel = chunk * incl
        p1.append(jnp.sum(sel[:, :E], axis=1, keepdims=True) - 1.0)
        p2.append(jnp.sum(sel[:, E:], axis=1, keepdims=True) - 1.0)
        base = base + jnp.sum(chunk, axis=0, keepdims=True)

    r1 = jnp.concatenate(p1, axis=0)
    r2 = (jnp.concatenate(p2, axis=0)
          + jnp.sum(oh2 * cnt1, axis=1, keepdims=True))
    pos1 = jnp.sum(oh1 * poff, axis=1, keepdims=True) + r1
    pos2 = jnp.sum(oh2 * poff, axis=1, keepdims=True) + r2
    pos1_ref[...] = pos1.astype(jnp.int32)
    pos2_ref[...] = pos2.astype(jnp.int32)

    bstart = (jax.lax.broadcasted_iota(jnp.int32, (128, 1), 0)
              * GB).astype(jnp.float32)
    nle = jnp.sum((poff <= bstart).astype(jnp.int32), axis=1, keepdims=True)
    be_ref[...] = jnp.clip(nle - 1, 0, E - 1)


def _router(x_flat, gate_w, gate_b, gate_temp):
    T = x_flat.shape[0]
    E = NUM_EXPERTS
    return pl.pallas_call(
        _router_body,
        out_shape=(
            jax.ShapeDtypeStruct((1, 1), jnp.float32),
            jax.ShapeDtypeStruct((T, 1), jnp.float32),
            jax.ShapeDtypeStruct((T, 1), jnp.float32),
            jax.ShapeDtypeStruct((T, 1), jnp.int32),
            jax.ShapeDtypeStruct((T, 1), jnp.int32),
            jax.ShapeDtypeStruct((128, 1), jnp.int32),
        ),
    )(x_flat, gate_w, gate_b.reshape(1, E), gate_temp.reshape(1, 1))


def _sc_scatter(x_flat, pos1, pos2):
    T, D = x_flat.shape
    TPW = T // NW
    mesh = plsc.VectorSubcoreMesh(core_axis_name="c", subcore_axis_name="s",
                                  num_cores=2, num_subcores=16)

    @functools.partial(
        pl.kernel, mesh=mesh,
        out_type=jax.ShapeDtypeStruct((P_PAD, D), jnp.float32),
        scratch_types=[
            pltpu.VMEM((TPW,), jnp.int32),
            pltpu.VMEM((TPW,), jnp.int32),
            pltpu.VMEM((TPW, D), jnp.float32),
            pltpu.SemaphoreType.DMA,
            pltpu.SemaphoreType.DMA,
        ],
    )
    def scatter_kernel(x_hbm, pos1_hbm, pos2_hbm, xs_hbm,
                       idx1_v, idx2_v, rows_v, sem1, sem2):
        wid = jax.lax.axis_index("s") * 2 + jax.lax.axis_index("c")
        base = wid * TPW
        pltpu.sync_copy(pos1_hbm.at[pl.ds(base, TPW)], idx1_v)
        pltpu.sync_copy(pos2_hbm.at[pl.ds(base, TPW)], idx2_v)
        pltpu.sync_copy(x_hbm.at[pl.ds(base, TPW)], rows_v)
        cp1 = pltpu.async_copy(rows_v, xs_hbm.at[idx1_v], sem1)
        cp2 = pltpu.async_copy(rows_v, xs_hbm.at[idx2_v], sem2)
        cp1.wait()
        cp2.wait()

    return scatter_kernel(x_flat, pos1, pos2)


def _sc_gather(y, pos1, pos2):
    T = pos1.shape[0]
    D = y.shape[1]
    TPW = T // NW
    mesh = plsc.VectorSubcoreMesh(core_axis_name="c", subcore_axis_name="s",
                                  num_cores=2, num_subcores=16)

    @functools.partial(
        pl.kernel, mesh=mesh,
        out_type=(
            jax.ShapeDtypeStruct((T, D), jnp.float32),
            jax.ShapeDtypeStruct((T, D), jnp.float32),
        ),
        scratch_types=[
            pltpu.VMEM((TPW,), jnp.int32),
            pltpu.VMEM((TPW, D), jnp.float32),
            pltpu.SemaphoreType.DMA,
        ],
    )
    def gather_kernel(y_hbm, pos1_hbm, pos2_hbm, g1_hbm, g2_hbm,
                      idx_v, rows_v, sem):
        wid = jax.lax.axis_index("s") * 2 + jax.lax.axis_index("c")
        base = wid * TPW
        pltpu.sync_copy(pos1_hbm.at[pl.ds(base, TPW)], idx_v)
        pltpu.async_copy(y_hbm.at[idx_v], rows_v, sem).wait()
        pltpu.sync_copy(rows_v, g1_hbm.at[pl.ds(base, TPW)])
        pltpu.sync_copy(pos2_hbm.at[pl.ds(base, TPW)], idx_v)
        pltpu.async_copy(y_hbm.at[idx_v], rows_v, sem).wait()
        pltpu.sync_copy(rows_v, g2_hbm.at[pl.ds(base, TPW)])

    return gather_kernel(y, pos1, pos2)


def _group_body(be_ref, fb_ref, par_ref, nxt_ref, xs_ref,
                w1_ref, w2_ref, b1_ref, b2_ref, y_ref,
                wbuf1, wbuf2, sem1, sem2):
    b = pl.program_id(0)
    e = be_ref[b]
    par = par_ref[b]

    @pl.when(b == 0)
    def _():
        pltpu.make_async_copy(w1_ref.at[e], wbuf1.at[par], sem1.at[par]).start()
        pltpu.make_async_copy(w2_ref.at[e], wbuf2.at[par], sem2.at[par]).start()

    @pl.when(fb_ref[b] == 1)
    def _():
        nxt = nxt_ref[b]

        @pl.when(nxt < NUM_EXPERTS)
        def _():
            opar = 1 - par
            pltpu.make_async_copy(w1_ref.at[nxt], wbuf1.at[opar],
                                  sem1.at[opar]).start()
            pltpu.make_async_copy(w2_ref.at[nxt], wbuf2.at[opar],
                                  sem2.at[opar]).start()

        pltpu.make_async_copy(w1_ref.at[e], wbuf1.at[par], sem1.at[par]).wait()
        pltpu.make_async_copy(w2_ref.at[e], wbuf2.at[par], sem2.at[par]).wait()

    h = jnp.maximum(
        jnp.dot(xs_ref[...], wbuf1[par], preferred_element_type=jnp.float32)
        + b1_ref[0], 0.0)
    y_ref[...] = (jnp.dot(h, wbuf2[par], preferred_element_type=jnp.float32)
                  + b2_ref[0])


def _grouped_ffn(be, fb, par, nxt, xs, w1, b1, w2, b2):
    D, F, E = D_MODEL, EXPERT_DIM, NUM_EXPERTS
    grid_spec = pltpu.PrefetchScalarGridSpec(
        num_scalar_prefetch=4,
        grid=(NB,),
        in_specs=[
            pl.BlockSpec((GB, D), lambda b, *_: (b, 0)),
            pl.BlockSpec(memory_space=pl.ANY),
            pl.BlockSpec(memory_space=pl.ANY),
            pl.BlockSpec((1, 1, F), lambda b, be, *_: (be[b], 0, 0)),
            pl.BlockSpec((1, 1, D), lambda b, be, *_: (be[b], 0, 0)),
        ],
        out_specs=pl.BlockSpec((GB, D), lambda b, *_: (b, 0)),
        scratch_shapes=[
            pltpu.VMEM((2, D, F), jnp.float32),
            pltpu.VMEM((2, F, D), jnp.float32),
            pltpu.SemaphoreType.DMA((2,)),
            pltpu.SemaphoreType.DMA((2,)),
        ],
    )
    return pl.pallas_call(
        _group_body,
        grid_spec=grid_spec,
        out_shape=jax.ShapeDtypeStruct((P_PAD, D), jnp.float32),
    )(be, fb, par, nxt, xs, w1, w2, b1.reshape(E, 1, F), b2.reshape(E, 1, D))


def _shared_body(x_ref, ws1_ref, bs1_ref, ws2_ref, bs2_ref, sh_ref):
    hs = jnp.maximum(
        jnp.dot(x_ref[...], ws1_ref[...], preferred_element_type=jnp.float32)
        + bs1_ref[...], 0.0)
    sh_ref[...] = (jnp.dot(hs, ws2_ref[...], preferred_element_type=jnp.float32)
                   + bs2_ref[...])


def _shared_ffn(x_flat, ws1, bs1, ws2, bs2):
    T, D = x_flat.shape
    F = EXPERT_DIM
    TB = 512
    return pl.pallas_call(
        _shared_body,
        grid=(T // TB,),
        in_specs=[
            pl.BlockSpec((TB, D), lambda t: (t, 0)),
            pl.BlockSpec((D, F), lambda t: (0, 0)),
            pl.BlockSpec((1, F), lambda t: (0, 0)),
            pl.BlockSpec((F, D), lambda t: (0, 0)),
            pl.BlockSpec((1, D), lambda t: (0, 0)),
        ],
        out_specs=pl.BlockSpec((TB, D), lambda t: (t, 0)),
        out_shape=jax.ShapeDtypeStruct((T, D), jnp.float32),
    )(x_flat, ws1, bs1.reshape(1, F), ws2, bs2.reshape(1, D))


def _combine_body(g1_ref, g2_ref, c1_ref, c2_ref, x_ref,
                  ws1_ref, bs1_ref, ws2_ref, bs2_ref, out_ref):
    hs = jnp.maximum(
        jnp.dot(x_ref[...], ws1_ref[...], preferred_element_type=jnp.float32)
        + bs1_ref[...], 0.0)
    shared = (jnp.dot(hs, ws2_ref[...], preferred_element_type=jnp.float32)
              + bs2_ref[...])
    out_ref[...] = (c1_ref[...] * g1_ref[...] + c2_ref[...] * g2_ref[...]
                    + shared)


def _combine(g1, g2, c1, c2, x_flat, ws1, bs1, ws2, bs2):
    T, D = x_flat.shape
    F = EXPERT_DIM
    TB = 512
    return pl.pallas_call(
        _combine_body,
        grid=(T // TB,),
        in_specs=[
            pl.BlockSpec((TB, D), lambda t: (t, 0)),
            pl.BlockSpec((TB, D), lambda t: (t, 0)),
            pl.BlockSpec((TB, 1), lambda t: (t, 0)),
            pl.BlockSpec((TB, 1), lambda t: (t, 0)),
            pl.BlockSpec((TB, D), lambda t: (t, 0)),
            pl.BlockSpec((D, F), lambda t: (0, 0)),
            pl.BlockSpec((1, F), lambda t: (0, 0)),
            pl.BlockSpec((F, D), lambda t: (0, 0)),
            pl.BlockSpec((1, D), lambda t: (0, 0)),
        ],
        out_specs=pl.BlockSpec((TB, D), lambda t: (t, 0)),
        out_shape=jax.ShapeDtypeStruct((T, D), jnp.float32),
    )(g1, g2, c1, c2, x_flat, ws1, bs1.reshape(1, F), ws2, bs2.reshape(1, D))


@jax.jit
def kernel(x, gate_w, gate_b, gate_temp, w1, b1, w2, b2, ws1, bs1, ws2, bs2):
    B, S, D = x.shape
    T = B * S
    x_flat = x.reshape(T, D)

    loss, c1, c2, pos1, pos2, be = _router(x_flat, gate_w, gate_b, gate_temp)
    pos1f = pos1.reshape(T)
    pos2f = pos2.reshape(T)
    be_nb = be.reshape(128)[:NB]

    idx = jnp.arange(NB, dtype=jnp.int32)
    sw = jnp.concatenate(
        [jnp.ones((1,), jnp.bool_), be_nb[1:] != be_nb[:-1]])
    fb = sw.astype(jnp.int32)
    par = (jnp.cumsum(fb) - 1) & 1
    swpos = jnp.where(sw, idx, NB)
    nxtpos = jnp.min(
        jnp.where(swpos[None, :] > idx[:, None], swpos[None, :], NB), axis=1)
    nxt = jnp.where(nxtpos < NB,
                    be_nb[jnp.clip(nxtpos, 0, NB - 1)], NUM_EXPERTS)
    nxt = nxt.astype(jnp.int32)

    xs = _sc_scatter(x_flat, pos1f, pos2f)
    y = _grouped_ffn(be_nb, fb, par, nxt, xs, w1, b1, w2, b2)
    g1, g2 = _sc_gather(y, pos1f, pos2f)
    out = _combine(g1, g2, c1, c2, x_flat, ws1, bs1, ws2, bs2)

    return out.reshape(B, S, D), loss.reshape(())


# speedup vs baseline: 1.1393x; 1.0028x over previous
import functools

import jax
import jax.numpy as jnp
from jax.experimental import pallas as pl
from jax.experimental.pallas import tpu as pltpu
from jax.experimental.pallas import tpu_sc as plsc

D_MODEL = 768
NUM_EXPERTS = 8
EXPERT_DIM = 2048
TOP_K = 2
EM_ITERS = 5
LOAD_BALANCE_WEIGHT = 0.1

GB = 256
NB = 39
P_PAD = NB * GB
NW = 32


def _router_body(x_ref, gw_ref, gb_ref, gt_ref,
                 loss_ref, c1_ref, c2_ref, pos1_ref, pos2_ref, be_ref):
    x = x_ref[...]
    T = x.shape[0]
    E = NUM_EXPERTS

    logits = jnp.dot(x, gw_ref[...], preferred_element_type=jnp.float32)
    logits = (logits + gb_ref[...]) / gt_ref[0, 0]

    m = jnp.max(logits, axis=1, keepdims=True)
    ex = jnp.exp(logits - m)
    sm = ex / jnp.sum(ex, axis=1, keepdims=True)

    iota = jax.lax.broadcasted_iota(jnp.int32, (T, E), 1)

    m1 = jnp.max(sm, axis=1, keepdims=True)
    i1 = jnp.min(jnp.where(sm == m1, iota, E), axis=1, keepdims=True)
    sm_masked = jnp.where(iota == i1, -jnp.inf, sm)
    m2 = jnp.max(sm_masked, axis=1, keepdims=True)
    i2 = jnp.min(jnp.where(sm_masked == m2, iota, E), axis=1, keepdims=True)

    oh1 = (iota == i1).astype(jnp.float32)
    oh2 = (iota == i2).astype(jnp.float32)

    usage = jnp.sum(oh1, axis=0, keepdims=True)
    actual = usage / jnp.float32(T) + 1e-8
    actual = actual / jnp.sum(actual)
    unif = jnp.float32(1.0 / E)
    kl = jnp.sum(unif * (jnp.log(unif) - jnp.log(actual)),
                 axis=1, keepdims=True)
    loss_ref[...] = LOAD_BALANCE_WEIGHT * kl

    p = jnp.full((1, E), 1.0 / E, dtype=jnp.float32)
    for _ in range(EM_ITERS):
        rs = jnp.dot(sm, p.reshape(E, 1),
                     preferred_element_type=jnp.float32)
        inv = 1.0 / (rs + 1e-8)
        counts = p * jnp.sum(sm * inv, axis=0, keepdims=True)
        p = counts / (jnp.sum(counts) + 1e-8)

    bal1 = m1 * jnp.sum(oh1 * p, axis=1, keepdims=True)
    bal2 = m2 * jnp.sum(oh2 * p, axis=1, keepdims=True)
    denom = bal1 + bal2 + 1e-8
    c1_ref[...] = bal1 / denom
    c2_ref[...] = bal2 / denom

    cnt1 = jnp.sum(oh1, axis=0, keepdims=True)
    cnt2 = jnp.sum(oh2, axis=0, keepdims=True)
    cnt = cnt1 + cnt2
    cnt_pad = jnp.ceil(cnt / GB) * GB

    r8 = jax.lax.broadcasted_iota(jnp.int32, (E, E), 0)
    c8 = jax.lax.broadcasted_iota(jnp.int32, (E, E), 1)
    poff = jnp.dot(cnt_pad, (r8 < c8).astype(jnp.float32),
                   preferred_element_type=jnp.float32)

    CH = 512
    tri = (jax.lax.broadcasted_iota(jnp.int32, (CH, CH), 0)
           >= jax.lax.broadcasted_iota(jnp.int32, (CH, CH), 1)
           ).astype(jnp.float32)

    ohj = jnp.concatenate([oh1, oh2], axis=1)
    base = jnp.zeros((1, 2 * E), jnp.float32)
    p1, p2 = [], []
    for ci in range(T // CH):
        chunk = ohj[ci * CH:(ci + 1) * CH]
        incl = jnp.dot(tri, chunk, preferred_element_type=jnp.float32) + base
        sel = chunk * incl
        p1.append(jnp.sum(sel[:, :E], axis=1, keepdims=True) - 1.0)
        p2.append(jnp.sum(sel[:, E:], axis=1, keepdims=True) - 1.0)
        base = base + jnp.sum(chunk, axis=0, keepdims=True)

    r1 = jnp.concatenate(p1, axis=0)
    r2 = (jnp.concatenate(p2, axis=0)
          + jnp.sum(oh2 * cnt1, axis=1, keepdims=True))
    pos1 = jnp.sum(oh1 * poff, axis=1, keepdims=True) + r1
    pos2 = jnp.sum(oh2 * poff, axis=1, keepdims=True) + r2
    pos1_ref[...] = pos1.astype(jnp.int32)
    pos2_ref[...] = pos2.astype(jnp.int32)

    bidx = jax.lax.broadcasted_iota(jnp.int32, (128, 1), 0)
    bstart = (bidx * GB).astype(jnp.float32)
    nle = jnp.sum((poff <= bstart).astype(jnp.int32), axis=1, keepdims=True)
    be = jnp.clip(nle - 1, 0, E - 1)

    be_prev = jnp.concatenate(
        [jnp.full((1, 1), -1, jnp.int32), be[:-1]], axis=0)
    sw = (be != be_prev).astype(jnp.float32)
    tri128 = (jax.lax.broadcasted_iota(jnp.int32, (128, 128), 0)
              >= jax.lax.broadcasted_iota(jnp.int32, (128, 128), 1)
              ).astype(jnp.float32)
    ordn = jnp.dot(tri128, sw, preferred_element_type=jnp.float32)
    par = (ordn.astype(jnp.int32) - 1) & 1
    swr = jnp.transpose(sw)
    idxr = jax.lax.broadcasted_iota(jnp.int32, (1, 128), 1)
    cand = jnp.where((idxr > bidx) & (swr > 0) & (idxr < NB), idxr, 128)
    nxtpos = jnp.min(cand, axis=1, keepdims=True)
    ber = jnp.transpose(be.astype(jnp.float32))
    nxt_val = jnp.sum(jnp.where(idxr == nxtpos, ber, 0.0),
                      axis=1, keepdims=True).astype(jnp.int32)
    nxt = jnp.where(nxtpos < NB, nxt_val, E)
    be_ref[...] = jnp.concatenate(
        [be, sw.astype(jnp.int32), par, nxt], axis=1)


def _router(x_flat, gate_w, gate_b, gate_temp):
    T = x_flat.shape[0]
    E = NUM_EXPERTS
    return pl.pallas_call(
        _router_body,
        out_shape=(
            jax.ShapeDtypeStruct((1, 1), jnp.float32),
            jax.ShapeDtypeStruct((T, 1), jnp.float32),
            jax.ShapeDtypeStruct((T, 1), jnp.float32),
            jax.ShapeDtypeStruct((T, 1), jnp.int32),
            jax.ShapeDtypeStruct((T, 1), jnp.int32),
            jax.ShapeDtypeStruct((128, 4), jnp.int32),
        ),
    )(x_flat, gate_w, gate_b.reshape(1, E), gate_temp.reshape(1, 1))


def _sc_scatter(x_flat, pos1, pos2):
    T, D = x_flat.shape
    TPW = T // NW
    mesh = plsc.VectorSubcoreMesh(core_axis_name="c", subcore_axis_name="s",
                                  num_cores=2, num_subcores=16)

    @functools.partial(
        pl.kernel, mesh=mesh,
        out_type=jax.ShapeDtypeStruct((P_PAD, D), jnp.float32),
        scratch_types=[
            pltpu.VMEM((TPW,), jnp.int32),
            pltpu.VMEM((TPW,), jnp.int32),
            pltpu.VMEM((TPW, D), jnp.float32),
            pltpu.SemaphoreType.DMA,
            pltpu.SemaphoreType.DMA,
        ],
    )
    def scatter_kernel(x_hbm, pos1_hbm, pos2_hbm, xs_hbm,
                       idx1_v, idx2_v, rows_v, sem1, sem2):
        wid = jax.lax.axis_index("s") * 2 + jax.lax.axis_index("c")
        base = wid * TPW
        pltpu.sync_copy(pos1_hbm.at[pl.ds(base, TPW)], idx1_v)
        pltpu.sync_copy(pos2_hbm.at[pl.ds(base, TPW)], idx2_v)
        pltpu.sync_copy(x_hbm.at[pl.ds(base, TPW)], rows_v)
        cp1 = pltpu.async_copy(rows_v, xs_hbm.at[idx1_v], sem1)
        cp2 = pltpu.async_copy(rows_v, xs_hbm.at[idx2_v], sem2)
        cp1.wait()
        cp2.wait()

    return scatter_kernel(x_flat, pos1, pos2)


def _sc_gather(y, pos1, pos2):
    T = pos1.shape[0]
    D = y.shape[1]
    TPW = T // NW
    mesh = plsc.VectorSubcoreMesh(core_axis_name="c", subcore_axis_name="s",
                                  num_cores=2, num_subcores=16)

    @functools.partial(
        pl.kernel, mesh=mesh,
        out_type=(
            jax.ShapeDtypeStruct((T, D), jnp.float32),
            jax.ShapeDtypeStruct((T, D), jnp.float32),
        ),
        scratch_types=[
            pltpu.VMEM((TPW,), jnp.int32),
            pltpu.VMEM((TPW, D), jnp.float32),
            pltpu.SemaphoreType.DMA,
        ],
    )
    def gather_kernel(y_hbm, pos1_hbm, pos2_hbm, g1_hbm, g2_hbm,
                      idx_v, rows_v, sem):
        wid = jax.lax.axis_index("s") * 2 + jax.lax.axis_index("c")
        base = wid * TPW
        pltpu.sync_copy(pos1_hbm.at[pl.ds(base, TPW)], idx_v)
        pltpu.async_copy(y_hbm.at[idx_v], rows_v, sem).wait()
        pltpu.sync_copy(rows_v, g1_hbm.at[pl.ds(base, TPW)])
        pltpu.sync_copy(pos2_hbm.at[pl.ds(base, TPW)], idx_v)
        pltpu.async_copy(y_hbm.at[idx_v], rows_v, sem).wait()
        pltpu.sync_copy(rows_v, g2_hbm.at[pl.ds(base, TPW)])

    return gather_kernel(y, pos1, pos2)


def _group_body(be_ref, fb_ref, par_ref, nxt_ref, xs_ref,
                w1_ref, w2_ref, b1_ref, b2_ref, y_ref,
                wbuf1, wbuf2, sem1, sem2):
    b = pl.program_id(0)
    e = be_ref[b]
    par = par_ref[b]

    @pl.when(b == 0)
    def _():
        pltpu.make_async_copy(w1_ref.at[e], wbuf1.at[par], sem1.at[par]).start()
        pltpu.make_async_copy(w2_ref.at[e], wbuf2.at[par], sem2.at[par]).start()

    @pl.when(fb_ref[b] == 1)
    def _():
        nxt = nxt_ref[b]

        @pl.when(nxt < NUM_EXPERTS)
        def _():
            opar = 1 - par
            pltpu.make_async_copy(w1_ref.at[nxt], wbuf1.at[opar],
                                  sem1.at[opar]).start()
            pltpu.make_async_copy(w2_ref.at[nxt], wbuf2.at[opar],
                                  sem2.at[opar]).start()

        pltpu.make_async_copy(w1_ref.at[e], wbuf1.at[par], sem1.at[par]).wait()
        pltpu.make_async_copy(w2_ref.at[e], wbuf2.at[par], sem2.at[par]).wait()

    h = jnp.maximum(
        jnp.dot(xs_ref[...], wbuf1[par], preferred_element_type=jnp.float32)
        + b1_ref[0], 0.0)
    y_ref[...] = (jnp.dot(h, wbuf2[par], preferred_element_type=jnp.float32)
                  + b2_ref[0])


def _grouped_ffn(be, fb, par, nxt, xs, w1, b1, w2, b2):
    D, F, E = D_MODEL, EXPERT_DIM, NUM_EXPERTS
    grid_spec = pltpu.PrefetchScalarGridSpec(
        num_scalar_prefetch=4,
        grid=(NB,),
        in_specs=[
            pl.BlockSpec((GB, D), lambda b, *_: (b, 0)),
            pl.BlockSpec(memory_space=pl.ANY),
            pl.BlockSpec(memory_space=pl.ANY),
            pl.BlockSpec((1, 1, F), lambda b, be, *_: (be[b], 0, 0)),
            pl.BlockSpec((1, 1, D), lambda b, be, *_: (be[b], 0, 0)),
        ],
        out_specs=pl.BlockSpec((GB, D), lambda b, *_: (b, 0)),
        scratch_shapes=[
            pltpu.VMEM((2, D, F), jnp.float32),
            pltpu.VMEM((2, F, D), jnp.float32),
            pltpu.SemaphoreType.DMA((2,)),
            pltpu.SemaphoreType.DMA((2,)),
        ],
    )
    return pl.pallas_call(
        _group_body,
        grid_spec=grid_spec,
        out_shape=jax.ShapeDtypeStruct((P_PAD, D), jnp.float32),
    )(be, fb, par, nxt, xs, w1, w2, b1.reshape(E, 1, F), b2.reshape(E, 1, D))


def _shared_body(x_ref, ws1_ref, bs1_ref, ws2_ref, bs2_ref, sh_ref):
    hs = jnp.maximum(
        jnp.dot(x_ref[...], ws1_ref[...], preferred_element_type=jnp.float32)
        + bs1_ref[...], 0.0)
    sh_ref[...] = (jnp.dot(hs, ws2_ref[...], preferred_element_type=jnp.float32)
                   + bs2_ref[...])


def _shared_ffn(x_flat, ws1, bs1, ws2, bs2):
    T, D = x_flat.shape
    F = EXPERT_DIM
    TB = 512
    return pl.pallas_call(
        _shared_body,
        grid=(T // TB,),
        in_specs=[
            pl.BlockSpec((TB, D), lambda t: (t, 0)),
            pl.BlockSpec((D, F), lambda t: (0, 0)),
            pl.BlockSpec((1, F), lambda t: (0, 0)),
            pl.BlockSpec((F, D), lambda t: (0, 0)),
            pl.BlockSpec((1, D), lambda t: (0, 0)),
        ],
        out_specs=pl.BlockSpec((TB, D), lambda t: (t, 0)),
        out_shape=jax.ShapeDtypeStruct((T, D), jnp.float32),
    )(x_flat, ws1, bs1.reshape(1, F), ws2, bs2.reshape(1, D))


def _combine_body(g1_ref, g2_ref, c1_ref, c2_ref, x_ref,
                  ws1_ref, bs1_ref, ws2_ref, bs2_ref, out_ref):
    hs = jnp.maximum(
        jnp.dot(x_ref[...], ws1_ref[...], preferred_element_type=jnp.float32)
        + bs1_ref[...], 0.0)
    shared = (jnp.dot(hs, ws2_ref[...], preferred_element_type=jnp.float32)
              + bs2_ref[...])
    out_ref[...] = (c1_ref[...] * g1_ref[...] + c2_ref[...] * g2_ref[...]
                    + shared)


def _combine(g1, g2, c1, c2, x_flat, ws1, bs1, ws2, bs2):
    T, D = x_flat.shape
    F = EXPERT_DIM
    TB = 512
    return pl.pallas_call(
        _combine_body,
        grid=(T // TB,),
        in_specs=[
            pl.BlockSpec((TB, D), lambda t: (t, 0)),
            pl.BlockSpec((TB, D), lambda t: (t, 0)),
            pl.BlockSpec((TB, 1), lambda t: (t, 0)),
            pl.BlockSpec((TB, 1), lambda t: (t, 0)),
            pl.BlockSpec((TB, D), lambda t: (t, 0)),
            pl.BlockSpec((D, F), lambda t: (0, 0)),
            pl.BlockSpec((1, F), lambda t: (0, 0)),
            pl.BlockSpec((F, D), lambda t: (0, 0)),
            pl.BlockSpec((1, D), lambda t: (0, 0)),
        ],
        out_specs=pl.BlockSpec((TB, D), lambda t: (t, 0)),
        out_shape=jax.ShapeDtypeStruct((T, D), jnp.float32),
    )(g1, g2, c1, c2, x_flat, ws1, bs1.reshape(1, F), ws2, bs2.reshape(1, D))


@jax.jit
def kernel(x, gate_w, gate_b, gate_temp, w1, b1, w2, b2, ws1, bs1, ws2, bs2):
    B, S, D = x.shape
    T = B * S
    x_flat = x.reshape(T, D)

    loss, c1, c2, pos1, pos2, pk = _router(x_flat, gate_w, gate_b, gate_temp)
    pos1f = pos1.reshape(T)
    pos2f = pos2.reshape(T)
    be_nb = pk[:NB, 0]
    fb = pk[:NB, 1]
    par = pk[:NB, 2]
    nxt = pk[:NB, 3]

    xs = _sc_scatter(x_flat, pos1f, pos2f)
    y = _grouped_ffn(be_nb, fb, par, nxt, xs, w1, b1, w2, b2)
    g1, g2 = _sc_gather(y, pos1f, pos2f)
    out = _combine(g1, g2, c1, c2, x_flat, ws1, bs1, ws2, bs2)

    return out.reshape(B, S, D), loss.reshape(())
